# baseline (speedup 1.0000x reference)
import functools

import jax
import jax.numpy as jnp
from jax import lax
from jax.experimental import pallas as pl
from jax.experimental.pallas import tpu as pltpu
from jax.experimental.pallas import tpu_sc as plsc

TOPK = 2
_N, _D, _H, _E = 2048, 768, 3072, 8
_M = 256
_MSH = 8
_NB = (_N * TOPK + _E * (_M - 1)) // _M
_P = _NB * _M
_NC, _NS = 2, 16
_NW = _NC * _NS
_TPW = _N // _NW
_L = 16


def _gelu_exact(a):
    return 0.5 * a * (1.0 + jax.lax.erf(a * (2.0 ** -0.5)))


def _gate_body(x_ref, gw_ref, gb_ref, e_ref, w_ref):
    l = jnp.dot(x_ref[...], gw_ref[...],
                preferred_element_type=jnp.float32) + gb_ref[...]
    n, e_dim = l.shape
    iota_e = jax.lax.broadcasted_iota(jnp.int32, (n, e_dim), 1)
    m1 = jnp.max(l, axis=1, keepdims=True)
    i1 = jnp.min(jnp.where(l == m1, iota_e, e_dim), axis=1, keepdims=True)
    lm = jnp.where(iota_e == i1, -jnp.inf, l)
    m2 = jnp.max(lm, axis=1, keepdims=True)
    i2 = jnp.min(jnp.where(lm == m2, iota_e, e_dim), axis=1, keepdims=True)
    b = jnp.exp(m2 - m1)
    w0 = 1.0 / (1.0 + b)
    w1 = 1.0 - w0
    e_ref[...] = jnp.concatenate([i1, i2], axis=1).astype(jnp.int32)
    w_ref[...] = jnp.concatenate([w0, w1], axis=1)


def _gate(x, gate_W, gate_b):
    n = x.shape[0]
    return pl.pallas_call(
        _gate_body,
        out_shape=(
            jax.ShapeDtypeStruct((n, TOPK), jnp.int32),
            jax.ShapeDtypeStruct((n, TOPK), jnp.float32),
        ),
    )(x, gate_W, gate_b.reshape(1, -1))


def _dispatch_body(e01_ref, x_ref, xg_ref, pos_ref, bexp_ref, nv_ref,
                   ev, hist, pre, p0v, p1v, xrows, bev, nvv, sem):
    wid = lax.axis_index("s") * _NC + lax.axis_index("c")
    iota = lax.iota(jnp.int32, _L)
    ones = jnp.ones((_L,), jnp.int32)
    zeros = jnp.zeros((_L,), jnp.int32)

    pltpu.sync_copy(e01_ref, ev)
    hist[...] = zeros
    pre[...] = zeros
    nchunks = _N * TOPK // _L
    mylimit = wid * (_TPW * TOPK // _L)

    def _hist_step(c, carry):
        vec = ev[pl.ds(c * _L, _L)]
        plsc.addupdate_scatter(hist, [vec], ones)
        m = jnp.broadcast_to(c < mylimit, (_L,))
        plsc.addupdate_scatter(pre, [vec], ones, mask=m)
        return carry

    lax.fori_loop(0, nchunks, _hist_step, 0)

    g = hist[...]
    caps = ((g + (_M - 1)) >> _MSH) << _MSH
    offs_incl = plsc.cumsum(caps)
    offs = offs_incl - caps
    tb = jnp.sum(caps) >> _MSH
    dest = offs + pre[...]

    for tc in range(_TPW // _L):
        base_flat = wid * _TPW * TOPK + tc * _L * TOPK
        for s in range(TOPK):
            evec = plsc.load_gather(ev, [base_flat + iota * TOPK + s])
            pos = zeros
            for e in range(_E):
                m = evec == e
                mi = m.astype(jnp.int32)
                prefix = plsc.cumsum(mi)
                de = jnp.sum(jnp.where(iota == e, dest, zeros))
                pos = jnp.where(m, de + prefix - 1, pos)
                dest = dest + jnp.where(iota == e, jnp.sum(mi), 0)
            if s == 0:
                p0v[pl.ds(tc * _L, _L)] = pos
            else:
                p1v[pl.ds(tc * _L, _L)] = pos

    pltpu.sync_copy(p0v, pos_ref.at[0, pl.ds(wid * _TPW, _TPW)])
    pltpu.sync_copy(p1v, pos_ref.at[1, pl.ds(wid * _TPW, _TPW)])

    pltpu.sync_copy(x_ref.at[pl.ds(wid * _TPW, _TPW)], xrows)
    pltpu.async_copy(xrows, xg_ref.at[p0v], sem).wait()
    pltpu.async_copy(xrows, xg_ref.at[p1v], sem).wait()

    @pl.when(wid == 0)
    def _():
        bs = offs >> _MSH
        for ch in range(2):
            bvec = jnp.minimum(iota + ch * _L, tb - 1)
            cnt = zeros
            for e in range(_E):
                bse = jnp.sum(jnp.where(iota == e, bs, zeros))
                cnt = cnt + (bvec >= bse).astype(jnp.int32)
            bev[pl.ds(ch * _L, _L)] = cnt - 1
        usedi = (caps > 0).astype(jnp.int32)
        rank_incl = plsc.cumsum(usedi)
        slotv = (rank_incl - 1) & 1
        nxtv = jnp.full((_L,), -1, jnp.int32)
        cur = jnp.int32(-1)
        for e in reversed(range(_E)):
            nxtv = jnp.where(iota == e, cur, nxtv)
            ue = jnp.sum(jnp.where(iota == e, usedi, zeros))
            cur = jnp.where(ue > 0, jnp.int32(e), cur)
        bev[pl.ds(2 * _L, _L)] = nxtv
        bev[pl.ds(3 * _L, _L)] = slotv
        nvv[...] = jnp.broadcast_to(tb, (_L,))
        pltpu.sync_copy(bev, bexp_ref)
        pltpu.sync_copy(nvv, nv_ref)


@functools.lru_cache(maxsize=1)
def _make_dispatch():
    return functools.partial(
        pl.kernel,
        out_type=(
            jax.ShapeDtypeStruct((_P, _D), jnp.float32),
            jax.ShapeDtypeStruct((TOPK, _N), jnp.int32),
            jax.ShapeDtypeStruct((4 * _L,), jnp.int32),
            jax.ShapeDtypeStruct((_L,), jnp.int32),
        ),
        mesh=plsc.VectorSubcoreMesh(core_axis_name="c", subcore_axis_name="s",
                                    num_cores=_NC, num_subcores=_NS),
        scratch_types=[
            pltpu.VMEM((_N * TOPK,), jnp.int32),
            pltpu.VMEM((_L,), jnp.int32),
            pltpu.VMEM((_L,), jnp.int32),
            pltpu.VMEM((_TPW,), jnp.int32),
            pltpu.VMEM((_TPW,), jnp.int32),
            pltpu.VMEM((_TPW, _D), jnp.float32),
            pltpu.VMEM((4 * _L,), jnp.int32),
            pltpu.VMEM((_L,), jnp.int32),
            pltpu.SemaphoreType.DMA,
        ],
        compiler_params=pltpu.CompilerParams(needs_layout_passes=False),
    )(_dispatch_body)


def _ffn_body(meta_ref, nv_ref, xg_ref, b1_ref, b2_ref, w1_hbm, w2_hbm, y_ref,
              w1buf, w2buf, sem1, sem2):
    b = pl.program_id(0)
    e_b = meta_ref[b]
    first = jnp.logical_or(b == 0, meta_ref[jnp.maximum(b - 1, 0)] != e_b)
    slot = meta_ref[3 * _L + e_b]
    nxt = meta_ref[2 * _L + e_b]
    nv = nv_ref[0]

    @pl.when(b == 0)
    def _():
        pltpu.make_async_copy(w1_hbm.at[e_b], w1buf.at[slot],
                              sem1.at[slot]).start()
        pltpu.make_async_copy(w2_hbm.at[e_b], w2buf.at[slot],
                              sem2.at[slot]).start()

    first_live = jnp.logical_and(first, b < nv)

    @pl.when(first_live)
    def _():
        pltpu.make_async_copy(w1_hbm.at[e_b], w1buf.at[slot],
                              sem1.at[slot]).wait()

    @pl.when(b < nv)
    def _():
        h = jnp.dot(xg_ref[...], w1buf[slot], preferred_element_type=jnp.float32)
        h = _gelu_exact(h + b1_ref[0])

        @pl.when(first_live)
        def _():
            pltpu.make_async_copy(w2_hbm.at[e_b], w2buf.at[slot],
                                  sem2.at[slot]).wait()

            @pl.when(nxt >= 0)
            def _():
                pltpu.make_async_copy(w1_hbm.at[nxt], w1buf.at[1 - slot],
                                      sem1.at[1 - slot]).start()
                pltpu.make_async_copy(w2_hbm.at[nxt], w2buf.at[1 - slot],
                                      sem2.at[1 - slot]).start()

        y_ref[...] = (jnp.dot(h, w2buf[slot], preferred_element_type=jnp.float32)
                      + b2_ref[0])


def _grouped_ffn(bexp, nv, xg, W1, b1, W2, b2):
    grid_spec = pltpu.PrefetchScalarGridSpec(
        num_scalar_prefetch=2,
        grid=(_NB,),
        in_specs=[
            pl.BlockSpec((_M, _D),
                         lambda b, be, nv: (jnp.minimum(b, nv[0] - 1), 0)),
            pl.BlockSpec((1, 1, _H), lambda b, be, nv: (be[b], 0, 0)),
            pl.BlockSpec((1, 1, _D), lambda b, be, nv: (be[b], 0, 0)),
            pl.BlockSpec(memory_space=pltpu.HBM),
            pl.BlockSpec(memory_space=pltpu.HBM),
        ],
        out_specs=pl.BlockSpec(
            (_M, _D), lambda b, be, nv: (jnp.minimum(b, nv[0] - 1), 0)),
        scratch_shapes=[
            pltpu.VMEM((2, _D, _H), jnp.float32),
            pltpu.VMEM((2, _H, _D), jnp.float32),
            pltpu.SemaphoreType.DMA((2,)),
            pltpu.SemaphoreType.DMA((2,)),
        ],
    )
    return pl.pallas_call(
        _ffn_body,
        grid_spec=grid_spec,
        out_shape=jax.ShapeDtypeStruct((_P, _D), jnp.float32),
        compiler_params=pltpu.CompilerParams(
            dimension_semantics=("arbitrary",),
        ),
    )(bexp, nv, xg, b1.reshape(_E, 1, _H), b2.reshape(_E, 1, _D), W1, W2)


def _combine_body(y_ref, pos_ref, wflat_ref, out_ref,
                  p0v, p1v, wv, r0, r1, orows, sem):
    wid = lax.axis_index("s") * _NC + lax.axis_index("c")
    hw = _TPW // 2
    for hc in range(2):
        tbase = wid * _TPW + hc * hw
        pltpu.sync_copy(pos_ref.at[0, pl.ds(tbase, hw)], p0v)
        pltpu.sync_copy(pos_ref.at[1, pl.ds(tbase, hw)], p1v)
        pltpu.sync_copy(wflat_ref.at[pl.ds(tbase * TOPK, hw * TOPK)], wv)
        pltpu.async_copy(y_ref.at[p0v], r0, sem).wait()
        pltpu.async_copy(y_ref.at[p1v], r1, sem).wait()

        def _row(r, carry):
            w0 = plsc.load_gather(wv, [jnp.full((_L,), 2 * r, jnp.int32)])
            w1 = plsc.load_gather(wv, [jnp.full((_L,), 2 * r + 1, jnp.int32)])
            for c in range(_D // _L):
                sl = pl.ds(c * _L, _L)
                orows[r, sl] = w0 * r0[r, sl] + w1 * r1[r, sl]
            return carry

        lax.fori_loop(0, hw, _row, 0)
        pltpu.sync_copy(orows, out_ref.at[pl.ds(tbase, hw)])


@functools.lru_cache(maxsize=1)
def _make_combine():
    return functools.partial(
        pl.kernel,
        out_type=jax.ShapeDtypeStruct((_N, _D), jnp.float32),
        mesh=plsc.VectorSubcoreMesh(core_axis_name="c", subcore_axis_name="s",
                                    num_cores=_NC, num_subcores=_NS),
        scratch_types=[
            pltpu.VMEM((_TPW // 2,), jnp.int32),
            pltpu.VMEM((_TPW // 2,), jnp.int32),
            pltpu.VMEM((_TPW,), jnp.float32),
            pltpu.VMEM((_TPW // 2, _D), jnp.float32),
            pltpu.VMEM((_TPW // 2, _D), jnp.float32),
            pltpu.VMEM((_TPW // 2, _D), jnp.float32),
            pltpu.SemaphoreType.DMA,
        ],
        compiler_params=pltpu.CompilerParams(needs_layout_passes=False),
    )(_combine_body)


def kernel(x, gate_W, gate_b, W1, b1, W2, b2):
    B, S, D = x.shape
    xf = x.reshape(-1, D)
    e01, w01 = _gate(xf, gate_W, gate_b)
    xg, pos, bexp, nv = _make_dispatch()(e01.reshape(-1), xf)
    y = _grouped_ffn(bexp, nv, xg, W1, b1, W2, b2)
    out = _make_combine()(y, pos, w01.reshape(-1))
    return out.reshape(B, S, D)


# speedup vs baseline: 1.1299x; 1.1299x over previous
import functools

import jax
import jax.numpy as jnp
from jax import lax
from jax.experimental import pallas as pl
from jax.experimental.pallas import tpu as pltpu
from jax.experimental.pallas import tpu_sc as plsc

TOPK = 2
_N, _D, _H, _E = 2048, 768, 3072, 8
_M = 256
_MSH = 8
_NB = (_N * TOPK + _E * (_M - 1)) // _M
_P = _NB * _M
_NC, _NS = 2, 16
_NW = _NC * _NS
_TPW = _N // _NW
_L = 16


def _gelu_exact(a):
    return 0.5 * a * (1.0 + jax.lax.erf(a * (2.0 ** -0.5)))


def _gate_body(x_ref, gw_ref, gb_ref, e_ref, w_ref):
    l = jnp.dot(x_ref[...], gw_ref[...],
                preferred_element_type=jnp.float32) + gb_ref[...]
    n, e_dim = l.shape
    iota_e = jax.lax.broadcasted_iota(jnp.int32, (n, e_dim), 1)
    m1 = jnp.max(l, axis=1, keepdims=True)
    i1 = jnp.min(jnp.where(l == m1, iota_e, e_dim), axis=1, keepdims=True)
    lm = jnp.where(iota_e == i1, -jnp.inf, l)
    m2 = jnp.max(lm, axis=1, keepdims=True)
    i2 = jnp.min(jnp.where(lm == m2, iota_e, e_dim), axis=1, keepdims=True)
    b = jnp.exp(m2 - m1)
    w0 = 1.0 / (1.0 + b)
    w1 = 1.0 - w0
    e_ref[...] = jnp.concatenate([i1, i2], axis=1).astype(jnp.int32)
    w_ref[...] = jnp.concatenate([w0, w1], axis=1)


def _gate(x, gate_W, gate_b):
    n = x.shape[0]
    return pl.pallas_call(
        _gate_body,
        out_shape=(
            jax.ShapeDtypeStruct((n, TOPK), jnp.int32),
            jax.ShapeDtypeStruct((n, TOPK), jnp.float32),
        ),
    )(x, gate_W, gate_b.reshape(1, -1))


def _dispatch_body(e01_ref, x_ref, xg_ref, pos_ref, bexp_ref, nv_ref,
                   ev, hist, pre, p0v, p1v, xrows, bev, nvv, sem, semb, semx):
    wid = lax.axis_index("s") * _NC + lax.axis_index("c")
    iota = lax.iota(jnp.int32, _L)
    ones = jnp.ones((_L,), jnp.int32)
    zeros = jnp.zeros((_L,), jnp.int32)

    xcp = pltpu.async_copy(x_ref.at[pl.ds(wid * _TPW, _TPW)], xrows, semx)
    pltpu.sync_copy(e01_ref, ev)
    hist[...] = zeros
    pre[...] = zeros
    nchunks = _N * TOPK // _L
    mylimit = wid * (_TPW * TOPK // _L)

    def _hist_step(c, carry):
        vec = ev[pl.ds(c * _L, _L)]
        plsc.addupdate_scatter(hist, [vec], ones)
        m = jnp.broadcast_to(c < mylimit, (_L,))
        plsc.addupdate_scatter(pre, [vec], ones, mask=m)
        return carry

    lax.fori_loop(0, nchunks, _hist_step, 0)

    g = hist[...]
    caps = ((g + (_M - 1)) >> _MSH) << _MSH
    offs_incl = plsc.cumsum(caps)
    offs = offs_incl - caps
    tb = jnp.sum(caps) >> _MSH
    dest = offs + pre[...]

    for tc in range(_TPW // _L):
        base_flat = wid * _TPW * TOPK + tc * _L * TOPK
        for s in range(TOPK):
            evec = plsc.load_gather(ev, [base_flat + iota * TOPK + s])
            pos = zeros
            for e in range(_E):
                m = evec == e
                mi = m.astype(jnp.int32)
                prefix = plsc.cumsum(mi)
                de = jnp.sum(jnp.where(iota == e, dest, zeros))
                pos = jnp.where(m, de + prefix - 1, pos)
                dest = dest + jnp.where(iota == e, jnp.sum(mi), 0)
            if s == 0:
                p0v[pl.ds(tc * _L, _L)] = pos
            else:
                p1v[pl.ds(tc * _L, _L)] = pos

    pltpu.sync_copy(p0v, pos_ref.at[0, pl.ds(wid * _TPW, _TPW)])
    pltpu.sync_copy(p1v, pos_ref.at[1, pl.ds(wid * _TPW, _TPW)])

    xcp.wait()
    c0 = pltpu.async_copy(xrows, xg_ref.at[p0v], sem)
    c1 = pltpu.async_copy(xrows, xg_ref.at[p1v], semb)
    c0.wait()
    c1.wait()

    @pl.when(wid == 0)
    def _():
        bs = offs >> _MSH
        for ch in range(2):
            bvec = jnp.minimum(iota + ch * _L, tb - 1)
            cnt = zeros
            for e in range(_E):
                bse = jnp.sum(jnp.where(iota == e, bs, zeros))
                cnt = cnt + (bvec >= bse).astype(jnp.int32)
            bev[pl.ds(ch * _L, _L)] = cnt - 1
        usedi = (caps > 0).astype(jnp.int32)
        rank_incl = plsc.cumsum(usedi)
        slotv = (rank_incl - 1) & 1
        nxtv = jnp.full((_L,), -1, jnp.int32)
        cur = jnp.int32(-1)
        for e in reversed(range(_E)):
            nxtv = jnp.where(iota == e, cur, nxtv)
            ue = jnp.sum(jnp.where(iota == e, usedi, zeros))
            cur = jnp.where(ue > 0, jnp.int32(e), cur)
        bev[pl.ds(2 * _L, _L)] = nxtv
        bev[pl.ds(3 * _L, _L)] = slotv
        nvv[...] = jnp.broadcast_to(tb, (_L,))
        pltpu.sync_copy(bev, bexp_ref)
        pltpu.sync_copy(nvv, nv_ref)


@functools.lru_cache(maxsize=1)
def _make_dispatch():
    return functools.partial(
        pl.kernel,
        out_type=(
            jax.ShapeDtypeStruct((_P, _D), jnp.float32),
            jax.ShapeDtypeStruct((TOPK, _N), jnp.int32),
            jax.ShapeDtypeStruct((4 * _L,), jnp.int32),
            jax.ShapeDtypeStruct((_L,), jnp.int32),
        ),
        mesh=plsc.VectorSubcoreMesh(core_axis_name="c", subcore_axis_name="s",
                                    num_cores=_NC, num_subcores=_NS),
        scratch_types=[
            pltpu.VMEM((_N * TOPK,), jnp.int32),
            pltpu.VMEM((_L,), jnp.int32),
            pltpu.VMEM((_L,), jnp.int32),
            pltpu.VMEM((_TPW,), jnp.int32),
            pltpu.VMEM((_TPW,), jnp.int32),
            pltpu.VMEM((_TPW, _D), jnp.float32),
            pltpu.VMEM((4 * _L,), jnp.int32),
            pltpu.VMEM((_L,), jnp.int32),
            pltpu.SemaphoreType.DMA,
            pltpu.SemaphoreType.DMA,
            pltpu.SemaphoreType.DMA,
        ],
        compiler_params=pltpu.CompilerParams(needs_layout_passes=False),
    )(_dispatch_body)


def _ffn_body(meta_ref, nv_ref, xg_ref, b1_ref, b2_ref, w1_hbm, w2_hbm, y_ref,
              w1buf, w2buf, sem1, sem2):
    b = pl.program_id(0)
    e_b = meta_ref[b]
    first = jnp.logical_or(b == 0, meta_ref[jnp.maximum(b - 1, 0)] != e_b)
    slot = meta_ref[3 * _L + e_b]
    nxt = meta_ref[2 * _L + e_b]
    nv = nv_ref[0]

    @pl.when(b == 0)
    def _():
        pltpu.make_async_copy(w1_hbm.at[e_b], w1buf.at[slot],
                              sem1.at[slot]).start()
        pltpu.make_async_copy(w2_hbm.at[e_b], w2buf.at[slot],
                              sem2.at[slot]).start()

    @pl.when(jnp.logical_and(first, b < nv))
    def _():
        pltpu.make_async_copy(w1_hbm.at[e_b], w1buf.at[slot],
                              sem1.at[slot]).wait()
        pltpu.make_async_copy(w2_hbm.at[e_b], w2buf.at[slot],
                              sem2.at[slot]).wait()

        @pl.when(nxt >= 0)
        def _():
            pltpu.make_async_copy(w1_hbm.at[nxt], w1buf.at[1 - slot],
                                  sem1.at[1 - slot]).start()
            pltpu.make_async_copy(w2_hbm.at[nxt], w2buf.at[1 - slot],
                                  sem2.at[1 - slot]).start()

    @pl.when(b < nv)
    def _():
        h = jnp.dot(xg_ref[...], w1buf[slot], preferred_element_type=jnp.float32)
        h = _gelu_exact(h + b1_ref[0])
        y_ref[...] = (jnp.dot(h, w2buf[slot], preferred_element_type=jnp.float32)
                      + b2_ref[0])


def _grouped_ffn(bexp, nv, xg, W1, b1, W2, b2):
    grid_spec = pltpu.PrefetchScalarGridSpec(
        num_scalar_prefetch=2,
        grid=(_NB,),
        in_specs=[
            pl.BlockSpec((_M, _D), lambda b, be, nv: (b, 0)),
            pl.BlockSpec((1, 1, _H), lambda b, be, nv: (be[b], 0, 0)),
            pl.BlockSpec((1, 1, _D), lambda b, be, nv: (be[b], 0, 0)),
            pl.BlockSpec(memory_space=pltpu.HBM),
            pl.BlockSpec(memory_space=pltpu.HBM),
        ],
        out_specs=pl.BlockSpec((_M, _D), lambda b, be, nv: (b, 0)),
        scratch_shapes=[
            pltpu.VMEM((2, _D, _H), jnp.float32),
            pltpu.VMEM((2, _H, _D), jnp.float32),
            pltpu.SemaphoreType.DMA((2,)),
            pltpu.SemaphoreType.DMA((2,)),
        ],
    )
    return pl.pallas_call(
        _ffn_body,
        grid_spec=grid_spec,
        out_shape=jax.ShapeDtypeStruct((_P, _D), jnp.float32),
        compiler_params=pltpu.CompilerParams(
            dimension_semantics=("arbitrary",),
        ),
    )(bexp, nv, xg, b1.reshape(_E, 1, _H), b2.reshape(_E, 1, _D), W1, W2)


def _combine_body(y_ref, pos_ref, wflat_ref, out_ref,
                  p0v, p1v, wv, r0, r1, orows, sem, semb):
    wid = lax.axis_index("s") * _NC + lax.axis_index("c")
    hw = _TPW // 2
    for hc in range(2):
        tbase = wid * _TPW + hc * hw
        pltpu.sync_copy(pos_ref.at[0, pl.ds(tbase, hw)], p0v)
        pltpu.sync_copy(pos_ref.at[1, pl.ds(tbase, hw)], p1v)
        pltpu.sync_copy(wflat_ref.at[pl.ds(tbase * TOPK, hw * TOPK)], wv)
        g0 = pltpu.async_copy(y_ref.at[p0v], r0, sem)
        g1 = pltpu.async_copy(y_ref.at[p1v], r1, semb)
        g0.wait()
        g1.wait()

        def _row(r, carry):
            w0 = plsc.load_gather(wv, [jnp.full((_L,), 2 * r, jnp.int32)])
            w1 = plsc.load_gather(wv, [jnp.full((_L,), 2 * r + 1, jnp.int32)])
            for c in range(_D // _L):
                sl = pl.ds(c * _L, _L)
                orows[r, sl] = w0 * r0[r, sl] + w1 * r1[r, sl]
            return carry

        lax.fori_loop(0, hw, _row, 0)
        pltpu.sync_copy(orows, out_ref.at[pl.ds(tbase, hw)])


@functools.lru_cache(maxsize=1)
def _make_combine():
    return functools.partial(
        pl.kernel,
        out_type=jax.ShapeDtypeStruct((_N, _D), jnp.float32),
        mesh=plsc.VectorSubcoreMesh(core_axis_name="c", subcore_axis_name="s",
                                    num_cores=_NC, num_subcores=_NS),
        scratch_types=[
            pltpu.VMEM((_TPW // 2,), jnp.int32),
            pltpu.VMEM((_TPW // 2,), jnp.int32),
            pltpu.VMEM((_TPW,), jnp.float32),
            pltpu.VMEM((_TPW // 2, _D), jnp.float32),
            pltpu.VMEM((_TPW // 2, _D), jnp.float32),
            pltpu.VMEM((_TPW // 2, _D), jnp.float32),
            pltpu.SemaphoreType.DMA,
            pltpu.SemaphoreType.DMA,
        ],
        compiler_params=pltpu.CompilerParams(needs_layout_passes=False),
    )(_combine_body)


def kernel(x, gate_W, gate_b, W1, b1, W2, b2):
    B, S, D = x.shape
    xf = x.reshape(-1, D)
    e01, w01 = _gate(xf, gate_W, gate_b)
    xg, pos, bexp, nv = _make_dispatch()(e01.reshape(-1), xf)
    y = _grouped_ffn(bexp, nv, xg, W1, b1, W2, b2)
    out = _make_combine()(y, pos, w01.reshape(-1))
    return out.reshape(B, S, D)
